# Initial kernel scaffold; baseline (speedup 1.0000x reference)
import jax
import jax.numpy as jnp
from jax.experimental import pallas as pl


def kernel(low_frequency, high_frequency, params):
    raise NotImplementedError("write your pallas kernel here")


# speedup vs baseline: 7.6457x; 7.6457x over previous
import functools

import jax
import jax.numpy as jnp
from jax.experimental import pallas as pl
from jax.experimental.pallas import tpu as pltpu

NODE = 4096
DIM = 64
INPUT = 8192
TOPK = 32
ALPHA = 3.0

_BN1 = 512
_BN2 = 512
_BN3 = 256


def _leaky(x):
    return jnp.where(x >= 0, x, 0.01 * x)


def _rne_bf16(x):
    b = jax.lax.bitcast_convert_type(x, jnp.uint32)
    lsb = (b >> 16) & jnp.uint32(1)
    rounded = (b + jnp.uint32(0x7FFF) + lsb) & jnp.uint32(0xFFFF0000)
    return jax.lax.bitcast_convert_type(rounded, jnp.float32)


def _k1_body(lo_ref, w_ref, x_ref):
    x_ref[...] = jax.lax.dot_general(
        lo_ref[...], w_ref[...], (((0,), (1,)), ((), ())),
        preferred_element_type=jnp.float32)


def _k1(inp2, w1):
    grid = (NODE // _BN1,)
    return pl.pallas_call(
        _k1_body,
        grid=grid,
        in_specs=[
            pl.BlockSpec((INPUT, _BN1), lambda n: (0, n)),
            pl.BlockSpec((128, INPUT), lambda n: (0, 0)),
        ],
        out_specs=pl.BlockSpec((_BN1, 128), lambda n: (n, 0)),
        out_shape=jax.ShapeDtypeStruct((NODE, 128), jnp.float32),
        compiler_params=pltpu.CompilerParams(
            dimension_semantics=("arbitrary",)),
    )(inp2, w1)


def _bn(x, g, b):
    m = jnp.mean(x, axis=0, keepdims=True)
    v = jnp.mean((x - m) ** 2, axis=0, keepdims=True)
    return g * (x - m) / jnp.sqrt(v + 1e-5) + b


def _k2a_body(xl_ref, xh_ref, w2_ref, b2_ref, g1_ref, be1_ref, g2_ref,
              be2_ref, h_ref):
    xl = xl_ref[...]
    xh = xh_ref[...]
    for i in range(4):
        x = (xl, xl, xh, xh)[i][:, (i % 2) * DIM:(i % 2) * DIM + DIM]
        h = _leaky(_bn(x, g1_ref[i], be1_ref[i]))
        h = jax.lax.dot_general(h, w2_ref[i], (((1,), (1,)), ((), ())),
                                preferred_element_type=jnp.float32)
        h = h + b2_ref[i]
        h = _leaky(_bn(h, g2_ref[i], be2_ref[i]))
        h_ref[:, i * DIM:(i + 1) * DIM] = h


def _k2a(xl, xh, w2, b2, g1, be1, g2, be2):
    return pl.pallas_call(
        _k2a_body,
        out_shape=jax.ShapeDtypeStruct((NODE, 4 * DIM), jnp.float32),
    )(xl, xh, w2, b2, g1, be1, g2, be2)


def _k2b_body(h_ref, w3_ref, b3_ref, bank_ref, v1_ref, v2_ref):
    nodes = []
    for i in range(4):
        h = h_ref[:, i * DIM:(i + 1) * DIM]
        logits = jax.lax.dot_general(
            h, w3_ref[i], (((1,), (1,)), ((), ())),
            preferred_element_type=jnp.float32) + b3_ref[i]
        m = jnp.max(logits, axis=1, keepdims=True)
        p = jnp.exp(logits - m)
        q = p / jnp.sum(p, axis=1, keepdims=True)
        bank = bank_ref[i]
        qhf = _rne_bf16(q)
        qlf = q - qhf
        bhf = _rne_bf16(bank)
        blf = bank - bhf
        dn = (((1,), (0,)), ((), ()))
        node = (jax.lax.dot_general(qhf, bhf, dn, preferred_element_type=jnp.float32)
                + jax.lax.dot_general(qhf, blf, dn, preferred_element_type=jnp.float32)
                + jax.lax.dot_general(qlf, bhf, dn, preferred_element_type=jnp.float32))
        nodes.append(_leaky(node))
    v1_ref[...] = 3.0 * nodes[0] + 3.0 * nodes[2]
    v2_ref[...] = 3.0 * nodes[1] + 3.0 * nodes[3]


def _k2b(h, w3, b3, banks):
    grid = (NODE // _BN2,)
    return pl.pallas_call(
        _k2b_body,
        grid=grid,
        in_specs=[
            pl.BlockSpec((_BN2, 4 * DIM), lambda b: (b, 0)),
            pl.BlockSpec((4, NODE, DIM), lambda b: (0, 0, 0)),
            pl.BlockSpec((4, 1, NODE), lambda b: (0, 0, 0)),
            pl.BlockSpec((4, NODE, DIM), lambda b: (0, 0, 0)),
        ],
        out_specs=[
            pl.BlockSpec((_BN2, DIM), lambda b: (b, 0)),
            pl.BlockSpec((_BN2, DIM), lambda b: (b, 0)),
        ],
        out_shape=[
            jax.ShapeDtypeStruct((NODE, DIM), jnp.float32),
            jax.ShapeDtypeStruct((NODE, DIM), jnp.float32),
        ],
        compiler_params=pltpu.CompilerParams(
            dimension_semantics=("parallel",)),
    )(h, w3, b3, banks)


def _k3_body(v1_ref, v2_ref, noise_ref, out_ref, scratch_ref):
    b = pl.program_id(0)
    v1r = v1_ref[pl.ds(b * _BN3, _BN3), :]
    v2r = v2_ref[pl.ds(b * _BN3, _BN3), :]
    dn = (((1,), (1,)), ((), ()))
    m1 = jax.lax.dot_general(v1r, v2_ref[...], dn,
                             preferred_element_type=jnp.float32)
    m2 = jax.lax.dot_general(v2r, v1_ref[...], dn,
                             preferred_element_type=jnp.float32)
    adj = jax.nn.relu(jnp.tanh(ALPHA * (m1 - m2)))
    w = adj + noise_ref[...]
    scratch_ref[...] = w

    def body(i, _):
        cur = scratch_ref[...]
        mx = jnp.max(cur, axis=1, keepdims=True)
        scratch_ref[...] = jnp.where(cur == mx, -1.0, cur)
        return mx

    t = jax.lax.fori_loop(0, TOPK, body, jnp.zeros((_BN3, 1), jnp.float32))
    out_ref[...] = jnp.where(w >= t, adj, 0.0)


def _k3(v1, v2, noise):
    grid = (NODE // _BN3,)
    return pl.pallas_call(
        _k3_body,
        grid=grid,
        in_specs=[
            pl.BlockSpec((NODE, DIM), lambda b: (0, 0)),
            pl.BlockSpec((NODE, DIM), lambda b: (0, 0)),
            pl.BlockSpec((_BN3, NODE), lambda b: (b, 0)),
        ],
        out_specs=pl.BlockSpec((_BN3, NODE), lambda b: (b, 0)),
        out_shape=jax.ShapeDtypeStruct((NODE, NODE), jnp.float32),
        scratch_shapes=[pltpu.VMEM((_BN3, NODE), jnp.float32)],
        compiler_params=pltpu.CompilerParams(
            dimension_semantics=("arbitrary",)),
    )(v1, v2, noise)


def kernel(low_frequency, high_frequency, params):
    e1, e2 = params["emb1"], params["emb2"]
    low2 = low_frequency.reshape(INPUT, NODE)
    high2 = high_frequency.reshape(INPUT, NODE)
    heads = (e1["ql"], e2["ql"], e1["qh"], e2["qh"])
    w1l = jnp.concatenate([heads[0]["W1"], heads[1]["W1"]], axis=0)
    w1h = jnp.concatenate([heads[2]["W1"], heads[3]["W1"]], axis=0)
    w2 = jnp.stack([h["W2"] for h in heads])
    b2 = jnp.stack([h["b2"] for h in heads])[:, None, :]
    g1 = jnp.stack([h["g1"] for h in heads])[:, None, :]
    be1 = jnp.stack([h["be1"] for h in heads])[:, None, :]
    g2 = jnp.stack([h["g2"] for h in heads])[:, None, :]
    be2 = jnp.stack([h["be2"] for h in heads])[:, None, :]
    w3 = jnp.stack([h["W3"] for h in heads])
    b3 = jnp.stack([h["b3"] for h in heads])[:, None, :]
    banks = jnp.stack([
        e1["low_bank"], e2["low_bank"], e1["high_bank"], e2["high_bank"]])

    xl = _k1(low2, w1l)
    xh = _k1(high2, w1h)
    h = _k2a(xl, xh, w2, b2, g1, be1, g2, be2)
    v1, v2 = _k2b(h, w3, b3, banks)
    noise = jax.random.uniform(jax.random.key(42), (NODE, NODE),
                               jnp.float32) * 0.01
    return _k3(v1, v2, noise)
